# baseline (speedup 1.0000x reference)
import functools

import jax
import jax.numpy as jnp
from jax import lax
from jax.experimental import pallas as pl
from jax.experimental.pallas import tpu as pltpu
from jax.experimental.pallas import tpu_sc as plsc

_B, _V = 128, 100000
_CH = 4000
_NCH = _V // _CH
_NW = 32
_RPW = _B // _NW
_GRP = _CH // 16
_UNROLL = 10

_CONST_CACHE = {}


def _gumbel_const():
    if "g" not in _CONST_CACHE:
        with jax.ensure_compile_time_eval():
            g = jax.random.gumbel(jax.random.key(42), (_B, _V), jnp.float32)
            _CONST_CACHE["g"] = g.reshape(-1)
    return _CONST_CACHE["g"]


def _permute(x, perm):
    return lax.gather(
        x, perm[:, None],
        lax.GatherDimensionNumbers(offset_dims=(), collapsed_slice_dims=(0,),
                                   start_index_map=(0,)),
        slice_sizes=(1,),
        mode=lax.GatherScatterMode.PROMISE_IN_BOUNDS)


def _sc_body(l_hbm, g_hbm, out_hbm,
             lbuf0, lbuf1, gbuf0, gbuf1, ansbuf,
             sl0, sl1, sg0, sg1):
    wid = lax.axis_index("s") * 2 + lax.axis_index("c")
    lane = lax.iota(jnp.int32, 16)
    lbufs, gbufs = (lbuf0, lbuf1), (gbuf0, gbuf1)
    sls, sgs = (sl0, sl1), (sg0, sg1)

    def start(chunk_id, p):
        sl = pl.ds(chunk_id * _CH, _CH)
        pltpu.make_async_copy(l_hbm.at[sl], lbufs[p], sls[p]).start()
        pltpu.make_async_copy(g_hbm.at[sl], gbufs[p], sgs[p]).start()

    def wait(p):
        sl = pl.ds(0, _CH)
        pltpu.make_async_copy(l_hbm.at[sl], lbufs[p], sls[p]).wait()
        pltpu.make_async_copy(g_hbm.at[sl], gbufs[p], sgs[p]).wait()

    def do_chunk(k, p, rc0, run_v, run_i, pre_ok):
        wait(p)

        def grp(i2, carry):
            rv, ri, colv = carry
            for u in range(_UNROLL):
                off = (i2 * _UNROLL + u) * 16
                phi = lbufs[p][pl.ds(off, 16)] + gbufs[p][pl.ds(off, 16)]
                better = phi > rv
                rv = jnp.where(better, phi, rv)
                ri = jnp.where(better, colv, ri)
                colv = colv + 16
            return rv, ri, colv

        colv0 = k * _CH + lane
        run_v, run_i, _ = lax.fori_loop(0, _GRP // _UNROLL, grp,
                                        (run_v, run_i, colv0))
        if pre_ok is not None:
            @pl.when(pre_ok)
            def _():
                start(rc0 + k + 2, p)
        return run_v, run_i

    ans = jnp.zeros((16,), jnp.int32)
    for j in range(_RPW):
        r = wid * _RPW + j
        rc0 = r * _NCH
        start(rc0 + 0, 0)
        start(rc0 + 1, 1)
        run_v = jnp.full((16,), -jnp.inf, jnp.float32)
        run_i = jnp.zeros((16,), jnp.int32)

        def chunk_pair(i, carry):
            rv, ri = carry
            t = jnp.bool_(True)
            rv, ri = do_chunk(2 * i, 0, rc0, rv, ri, t)
            rv, ri = do_chunk(2 * i + 1, 1, rc0, rv, ri, i < (_NCH - 3) // 2)
            return rv, ri

        run_v, run_i = lax.fori_loop(0, _NCH // 2, chunk_pair, (run_v, run_i))
        run_v, run_i = do_chunk(_NCH - 1, 0, rc0, run_v, run_i, None)

        rv, ri = run_v, run_i
        for s in (8, 4, 2, 1):
            perm = lane ^ s
            pv = _permute(rv, perm)
            pi = _permute(ri, perm)
            take = (pv > rv) | ((pv == rv) & (pi < ri))
            rv = jnp.where(take, pv, rv)
            ri = jnp.where(take, pi, ri)
        ans = jnp.where(lane == j, ri, ans)

    ansbuf[...] = ans
    pltpu.sync_copy(ansbuf, out_hbm.at[wid])


@functools.partial(
    pl.kernel,
    mesh=plsc.VectorSubcoreMesh(core_axis_name="c", subcore_axis_name="s"),
    out_type=jax.ShapeDtypeStruct((_NW, 16), jnp.int32),
    scratch_types=[
        pltpu.VMEM((_CH,), jnp.float32),
        pltpu.VMEM((_CH,), jnp.float32),
        pltpu.VMEM((_CH,), jnp.float32),
        pltpu.VMEM((_CH,), jnp.float32),
        pltpu.VMEM((16,), jnp.int32),
        pltpu.SemaphoreType.DMA,
        pltpu.SemaphoreType.DMA,
        pltpu.SemaphoreType.DMA,
        pltpu.SemaphoreType.DMA,
    ],
)
def _sc_sample(l_hbm, g_hbm, out_hbm, *rest):
    _sc_body(l_hbm, g_hbm, out_hbm, *rest)


@jax.jit
def _run(logits, g2):
    l2 = logits.reshape(-1)
    out = _sc_sample(l2, g2)
    return out[:, :_RPW].reshape(_B)


def kernel(logits):
    return _run(logits, _gumbel_const())


# speedup vs baseline: 2.6922x; 2.6922x over previous
import jax
import jax.numpy as jnp
from jax.experimental import pallas as pl
from jax.experimental.pallas import tpu as pltpu

_B, _V = 128, 100000
_BC = 16384
_NB = (_V + _BC - 1) // _BC
_K = _BC // 128

_CONST_CACHE = {}


def _gumbel_const():
    if "g" not in _CONST_CACHE:
        with jax.ensure_compile_time_eval():
            _CONST_CACHE["g"] = jax.random.gumbel(
                jax.random.key(42), (_B, _V), jnp.float32)
    return _CONST_CACHE["g"]


def _argmax_body(l_ref, g_ref, o_ref, vmax_ref, vidx_ref):
    b = pl.program_id(0)
    lane = jax.lax.broadcasted_iota(jnp.int32, (_B, 128), 1)
    neg_inf = jnp.float32(-jnp.inf)

    run_v = None
    for k in range(_K):
        sl = pl.ds(k * 128, 128)
        chunk = l_ref[:, sl] + g_ref[:, sl]
        col = lane + (b * _BC + k * 128)
        chunk = jnp.where(col < _V, chunk, neg_inf)
        if run_v is None:
            run_v, run_i = chunk, col
        else:
            better = chunk > run_v
            run_v = jnp.where(better, chunk, run_v)
            run_i = jnp.where(better, col, run_i)

    @pl.when(b == 0)
    def _():
        vmax_ref[...] = run_v
        vidx_ref[...] = run_i

    @pl.when(b > 0)
    def _():
        pv = vmax_ref[...]
        pi = vidx_ref[...]
        better = run_v > pv
        vmax_ref[...] = jnp.where(better, run_v, pv)
        vidx_ref[...] = jnp.where(better, run_i, pi)

    @pl.when(b == _NB - 1)
    def _():
        fv = vmax_ref[...]
        fi = vidx_ref[...]
        m = jnp.max(fv, axis=1, keepdims=True)
        cand = jnp.where(fv == m, fi, _V)
        o_ref[...] = jnp.min(cand, axis=1, keepdims=True)


@jax.jit
def _sample(logits, g):
    out = pl.pallas_call(
        _argmax_body,
        grid=(_NB,),
        in_specs=[pl.BlockSpec((_B, _BC), lambda b: (0, b)),
                  pl.BlockSpec((_B, _BC), lambda b: (0, b))],
        out_specs=pl.BlockSpec((_B, 1), lambda b: (0, 0)),
        out_shape=jax.ShapeDtypeStruct((_B, 1), jnp.int32),
        scratch_shapes=[pltpu.VMEM((_B, 128), jnp.float32),
                        pltpu.VMEM((_B, 128), jnp.int32)],
        compiler_params=pltpu.CompilerParams(
            dimension_semantics=("arbitrary",)),
    )(logits, g)
    return out[:, 0]


def kernel(logits):
    return _sample(logits, _gumbel_const())
